# baseline (speedup 1.0000x reference)
import functools

import jax
import jax.numpy as jnp
from jax import lax
from jax.experimental import pallas as pl
from jax.experimental.pallas import tpu as pltpu
from jax.experimental.pallas import tpu_sc as plsc

f32 = jnp.float32
i32 = jnp.int32

NC = 2
NS = 16
NW = NC * NS
C = 128


def _gather_kernel(N, DW, NCH):
    mesh = plsc.VectorSubcoreMesh(core_axis_name="c", subcore_axis_name="s",
                                  num_cores=NC, num_subcores=NS)

    @functools.partial(
        pl.kernel,
        out_type=jax.ShapeDtypeStruct((NW * NCH * C, DW), f32),
        mesh=mesh,
        scratch_types=[
            pltpu.VMEM((NCH, C), i32),
            pltpu.VMEM((C, DW), f32),
            pltpu.VMEM((C, DW), f32),
            pltpu.VMEM((C, DW), f32),
            pltpu.VMEM((C, DW), f32),
            pltpu.SemaphoreType.DMA,
            pltpu.SemaphoreType.DMA,
            pltpu.SemaphoreType.DMA,
            pltpu.SemaphoreType.DMA,
        ],
    )
    def gk(x_hbm, src_hbm, out_hbm, idx_v, r0, r1, r2, r3, g0, g1, g2, g3):
        wid = lax.axis_index("s") * NC + lax.axis_index("c")
        pltpu.sync_copy(src_hbm.at[wid], idx_v)
        rows = (r0, r1, r2, r3)
        sems = (g0, g1, g2, g3)

        def body(t, carry):
            j = 4 * t
            ds = [pltpu.async_copy(x_hbm.at[idx_v.at[j + i]], rows[i], sems[i])
                  for i in range(4)]
            for i in range(4):
                ds[i].wait()
                pltpu.sync_copy(rows[i],
                                out_hbm.at[pl.ds((wid * NCH + j + i) * C, C)])
            return carry

        lax.fori_loop(0, NCH // 4, body, 0)

    return gk


def _scatter_kernel(NP, DM, NCH, RPS):
    mesh = plsc.VectorSubcoreMesh(core_axis_name="c", subcore_axis_name="s",
                                  num_cores=NC, num_subcores=NS)

    @functools.partial(
        pl.kernel,
        out_type=jax.ShapeDtypeStruct((NC * NP, DM), f32),
        mesh=mesh,
        compiler_params=pltpu.CompilerParams(use_tc_tiling_on_sc=False),
        scratch_types=[
            pltpu.VMEM((NCH, C), i32),
            pltpu.VMEM((C, DM), f32),
            pltpu.VMEM((C, DM), f32),
            pltpu.VMEM((C, DM), f32),
            pltpu.VMEM((C, DM), f32),
            pltpu.VMEM_SHARED((NP, DM), f32),
            pltpu.SemaphoreType.DMA,
            pltpu.SemaphoreType.DMA,
            pltpu.SemaphoreType.DMA,
            pltpu.SemaphoreType.DMA,
            pltpu.SemaphoreType.DMA,
            pltpu.SemaphoreType.DMA,
            pltpu.SemaphoreType.DMA,
            pltpu.SemaphoreType.DMA,
        ],
    )
    def sk(msg_hbm, dst_hbm, z_hbm, agg_hbm,
           idx_v, r0, r1, r2, r3, agg_s,
           l0, l1, l2, l3, a0, a1, a2, a3):
        cid = lax.axis_index("c")
        sid = lax.axis_index("s")
        wid = sid * NC + cid

        pltpu.sync_copy(z_hbm, agg_s.at[pl.ds(sid * RPS, RPS)])
        pltpu.sync_copy(dst_hbm.at[wid], idx_v)
        plsc.subcore_barrier()

        rows = (r0, r1, r2, r3)
        lsem = (l0, l1, l2, l3)
        asem = (a0, a1, a2, a3)

        def body(t, carry):
            j = 4 * t
            lds = [pltpu.async_copy(
                msg_hbm.at[pl.ds((wid * NCH + j + i) * C, C)], rows[i],
                lsem[i]) for i in range(4)]
            sca = []
            for i in range(4):
                lds[i].wait()
                sca.append(pltpu.async_copy(
                    rows[i], agg_s.at[idx_v.at[j + i]], asem[i], add=True))
            for i in range(4):
                sca[i].wait()
            return carry

        lax.fori_loop(0, NCH // 4, body, 0)
        plsc.subcore_barrier()

        base = cid * NP + sid * RPS
        pltpu.sync_copy(agg_s.at[pl.ds(sid * RPS, RPS)],
                        agg_hbm.at[pl.ds(base, RPS)])

    return sk


def _msg_body(ED, D, HNN, DM, ea_ref, xj_ref, w1t_ref, b1_ref, w2ct_ref,
              out_ref, u_ref):
    be = ea_ref.shape[0]
    eaT = ea_ref[...].T
    he = jnp.dot(w1t_ref[...], eaT, preferred_element_type=f32)
    he = jnp.maximum(he + b1_ref[...], 0.0)
    xjT = xj_ref[:, :D].T
    for h in range(HNN):
        u_ref[h * D:(h + 1) * D, :] = (xjT * he[h:h + 1, :]).astype(
            jnp.bfloat16)
    u_ref[HNN * D:HNN * D + D, :] = xjT.astype(jnp.bfloat16)
    msgT = jnp.dot(w2ct_ref[...], u_ref[...], preferred_element_type=f32)
    out_ref[...] = jnp.concatenate(
        [msgT.T, jnp.ones((be, 1), f32), jnp.zeros((be, DM - D - 1), f32)],
        axis=1)


def _tail_body(D, x_ref, agg0_ref, agg1_ref, wr_ref,
               bc_ref, wih_ref, bih_ref, whh_ref, bhh_ref, xo_ref, hn_ref):
    xb = x_ref[...]
    s = agg0_ref[...] + agg1_ref[...]
    c = s[:, D:D + 1]
    mean = s[:, :D] / jnp.maximum(c, 1.0)
    conv = mean + jnp.dot(xb, wr_ref[...], preferred_element_type=f32)
    conv = conv + bc_ref[...]
    xin = jnp.maximum(conv, 0.0) + (jnp.exp(jnp.minimum(conv, 0.0)) - 1.0)
    gi = jnp.dot(xin, wih_ref[...], preferred_element_type=f32) + bih_ref[...]
    gh = jnp.dot(xb, whh_ref[...], preferred_element_type=f32) + bhh_ref[...]
    r = jax.nn.sigmoid(gi[:, :D] + gh[:, :D])
    z = jax.nn.sigmoid(gi[:, D:2 * D] + gh[:, D:2 * D])
    n = jnp.tanh(gi[:, 2 * D:] + r * gh[:, 2 * D:])
    hn = (1.0 - z) * n + z * xb
    hn_ref[...] = hn
    xo_ref[...] = jnp.maximum(hn + xb, 0.0)


def kernel(x, edge_index, edge_attr, W1, b1, W2, b2, W_root, b_conv,
           W_ih, W_hh, b_ih, b_hh):
    N, D = x.shape
    E = edge_index.shape[1]
    ED = edge_attr.shape[1]
    HNN = W1.shape[1]

    NCH = -(-E // (NW * C))
    EP = NW * NCH * C
    DM = 2 * D
    RPS = ((-(-(N + 1) // NS)) + 7) // 8 * 8
    NP = NS * RPS
    BN = NP // 8
    pad = EP - E

    src = edge_index[0]
    dst = edge_index[1]
    src_p = jnp.concatenate([src, jnp.zeros((pad,), i32)]).reshape(NW, NCH, C)
    dst_p = jnp.concatenate([dst, jnp.full((pad,), N, i32)]).reshape(NW, NCH, C)

    DW = 128
    x128 = jnp.concatenate([x, jnp.zeros((N, DW - D), f32)], axis=1)
    x_j = _gather_kernel(N, DW, NCH)(x128, src_p)

    W1T = W1.T
    b1c = b1[:, None]
    W2c = W2.reshape(HNN, D, D).transpose(2, 0, 1).reshape(D, HNN * D)
    W2full = jnp.concatenate([W2c, b2.reshape(D, D).T], axis=1)

    BE = 1280
    n_blk = E // BE
    msg_p = pl.pallas_call(
        functools.partial(_msg_body, ED, D, HNN, DM),
        grid=(n_blk,),
        in_specs=[
            pl.BlockSpec((BE, ED), lambda i: (i, 0)),
            pl.BlockSpec((BE, DW), lambda i: (i, 0)),
            pl.BlockSpec((HNN, ED), lambda i: (0, 0)),
            pl.BlockSpec((HNN, 1), lambda i: (0, 0)),
            pl.BlockSpec((D, HNN * D + D), lambda i: (0, 0)),
        ],
        out_specs=pl.BlockSpec((BE, DM), lambda i: (i, 0)),
        out_shape=jax.ShapeDtypeStruct((EP, DM), f32),
        scratch_shapes=[pltpu.VMEM((HNN * D + D, BE), jnp.bfloat16)],
    )(edge_attr, x_j, W1T, b1c, W2full.astype(jnp.bfloat16))

    z = jnp.zeros((RPS, DM), f32)
    agg_out = _scatter_kernel(NP, DM, NCH, RPS)(msg_p, dst_p, z)

    x_pad = jnp.concatenate([x, jnp.zeros((NP - N, D), f32)])
    npb = NP // BN
    x_out, h_new = pl.pallas_call(
        functools.partial(_tail_body, D),
        grid=(8,),
        in_specs=[
            pl.BlockSpec((BN, D), lambda i: (i, 0)),
            pl.BlockSpec((BN, DM), lambda i: (i, 0)),
            pl.BlockSpec((BN, DM), lambda i: (i + npb, 0)),
            pl.BlockSpec((D, D), lambda i: (0, 0)),
            pl.BlockSpec((1, D), lambda i: (0, 0)),
            pl.BlockSpec((D, 3 * D), lambda i: (0, 0)),
            pl.BlockSpec((1, 3 * D), lambda i: (0, 0)),
            pl.BlockSpec((D, 3 * D), lambda i: (0, 0)),
            pl.BlockSpec((1, 3 * D), lambda i: (0, 0)),
        ],
        out_specs=[
            pl.BlockSpec((BN, D), lambda i: (i, 0)),
            pl.BlockSpec((BN, D), lambda i: (i, 0)),
        ],
        out_shape=[
            jax.ShapeDtypeStruct((NP, D), f32),
            jax.ShapeDtypeStruct((NP, D), f32),
        ],
    )(x_pad, agg_out, agg_out, W_root, b_conv[None, :],
      W_ih.T, b_ih[None, :], W_hh.T, b_hh[None, :])

    return (x_out[:N], h_new[None, :N, :])


# speedup vs baseline: 1.4412x; 1.4412x over previous
import functools

import jax
import jax.numpy as jnp
from jax import lax
from jax.experimental import pallas as pl
from jax.experimental.pallas import tpu as pltpu
from jax.experimental.pallas import tpu_sc as plsc

f32 = jnp.float32
bf16 = jnp.bfloat16
i32 = jnp.int32

NC = 2
NS = 16
NW = NC * NS
C = 128


def _gather_kernel(N, D, NCH):
    mesh = plsc.VectorSubcoreMesh(core_axis_name="c", subcore_axis_name="s",
                                  num_cores=NC, num_subcores=NS)

    @functools.partial(
        pl.kernel,
        out_type=jax.ShapeDtypeStruct((NW * NCH * C, D), bf16),
        mesh=mesh,
        compiler_params=pltpu.CompilerParams(use_tc_tiling_on_sc=False),
        scratch_types=[
            pltpu.VMEM((NCH, C), i32),
            pltpu.VMEM((C, D), bf16),
            pltpu.VMEM((C, D), bf16),
            pltpu.VMEM((C, D), bf16),
            pltpu.VMEM((C, D), bf16),
            pltpu.SemaphoreType.DMA,
            pltpu.SemaphoreType.DMA,
            pltpu.SemaphoreType.DMA,
            pltpu.SemaphoreType.DMA,
        ],
    )
    def gk(x_hbm, src_hbm, out_hbm, idx_v, r0, r1, r2, r3, g0, g1, g2, g3):
        wid = lax.axis_index("s") * NC + lax.axis_index("c")
        pltpu.sync_copy(src_hbm.at[wid], idx_v)
        rows = (r0, r1, r2, r3)
        sems = (g0, g1, g2, g3)

        def body(t, carry):
            j = 4 * t
            ds = [pltpu.async_copy(x_hbm.at[idx_v.at[j + i]], rows[i], sems[i])
                  for i in range(4)]
            for i in range(4):
                ds[i].wait()
                pltpu.sync_copy(rows[i],
                                out_hbm.at[pl.ds((wid * NCH + j + i) * C, C)])
            return carry

        lax.fori_loop(0, NCH // 4, body, 0)

    return gk


def _scatter_kernel(NP, D, NCH, RPS):
    mesh = plsc.VectorSubcoreMesh(core_axis_name="c", subcore_axis_name="s",
                                  num_cores=NC, num_subcores=NS)

    @functools.partial(
        pl.kernel,
        out_type=(
            jax.ShapeDtypeStruct((NC * NP, D), f32),
            jax.ShapeDtypeStruct((NC * NP, 16), f32),
        ),
        mesh=mesh,
        compiler_params=pltpu.CompilerParams(use_tc_tiling_on_sc=False),
        scratch_types=[
            pltpu.VMEM((NCH, C), i32),
            pltpu.VMEM((C, D), f32),
            pltpu.VMEM((C, D), f32),
            pltpu.VMEM((C, D), f32),
            pltpu.VMEM((C, D), f32),
            pltpu.VMEM((C, 16), f32),
            pltpu.VMEM_SHARED((NP, D), f32),
            pltpu.VMEM_SHARED((NP, 16), f32),
            pltpu.SemaphoreType.DMA,
            pltpu.SemaphoreType.DMA,
            pltpu.SemaphoreType.DMA,
            pltpu.SemaphoreType.DMA,
            pltpu.SemaphoreType.DMA,
            pltpu.SemaphoreType.DMA,
            pltpu.SemaphoreType.DMA,
            pltpu.SemaphoreType.DMA,
        ],
    )
    def sk(msg_hbm, dst_hbm, z32_hbm, z16_hbm, agg_hbm, cnt_hbm,
           idx_v, r0, r1, r2, r3, ones_v, agg_s, cnt_s,
           l0, l1, l2, l3, a0, a1, a2, a3):
        cid = lax.axis_index("c")
        sid = lax.axis_index("s")
        wid = sid * NC + cid

        o16 = jnp.ones((16,), f32)

        def fo(i, carry):
            ones_v[i, :] = o16
            return carry

        lax.fori_loop(0, C, fo, 0)

        pltpu.sync_copy(z32_hbm, agg_s.at[pl.ds(sid * RPS, RPS)])
        pltpu.sync_copy(z16_hbm, cnt_s.at[pl.ds(sid * RPS, RPS)])
        pltpu.sync_copy(dst_hbm.at[wid], idx_v)
        plsc.subcore_barrier()

        rows = (r0, r1, r2, r3)
        lsem = (l0, l1, l2, l3)
        asem = (a0, a1, a2, a3)

        def body(t, carry):
            j = 4 * t
            lds = [pltpu.async_copy(
                msg_hbm.at[pl.ds((wid * NCH + j + i) * C, C)], rows[i],
                lsem[i]) for i in range(4)]
            sca = []
            for i in range(4):
                lds[i].wait()
                sca.append(pltpu.async_copy(
                    rows[i], agg_s.at[idx_v.at[j + i]], asem[i], add=True))
                pltpu.sync_copy(ones_v, cnt_s.at[idx_v.at[j + i]], add=True)
            for i in range(4):
                sca[i].wait()
            return carry

        lax.fori_loop(0, NCH // 4, body, 0)
        plsc.subcore_barrier()

        base = cid * NP + sid * RPS
        pltpu.sync_copy(agg_s.at[pl.ds(sid * RPS, RPS)],
                        agg_hbm.at[pl.ds(base, RPS)])
        pltpu.sync_copy(cnt_s.at[pl.ds(sid * RPS, RPS)],
                        cnt_hbm.at[pl.ds(base, RPS)])

    return sk


def _msg_body(ED, D, HNN, ea_ref, xj_ref, w1t_ref, b1_ref, w2ct_ref,
              out_ref, u_ref):
    be = ea_ref.shape[0]
    eaT = ea_ref[...].T
    he = jnp.dot(w1t_ref[...], eaT, preferred_element_type=f32)
    he = jnp.maximum(he + b1_ref[...], 0.0)
    xjT = xj_ref[...].astype(f32).T
    for h in range(HNN):
        u_ref[h * D:(h + 1) * D, :] = (xjT * he[h:h + 1, :]).astype(bf16)
    u_ref[HNN * D:HNN * D + D, :] = xjT.astype(bf16)
    msgT = jnp.dot(w2ct_ref[...], u_ref[...], preferred_element_type=f32)
    del be
    out_ref[...] = msgT.T


def _tail_body(D, x_ref, agg0_ref, agg1_ref, cnt0_ref, cnt1_ref, wr_ref,
               bc_ref, wih_ref, bih_ref, whh_ref, bhh_ref, xo_ref, hn_ref):
    xb = x_ref[...]
    s = agg0_ref[...] + agg1_ref[...]
    c = cnt0_ref[:, 0:1] + cnt1_ref[:, 0:1]
    mean = s / jnp.maximum(c, 1.0)
    conv = mean + jnp.dot(xb, wr_ref[...], preferred_element_type=f32)
    conv = conv + bc_ref[...]
    xin = jnp.maximum(conv, 0.0) + (jnp.exp(jnp.minimum(conv, 0.0)) - 1.0)
    gi = jnp.dot(xin, wih_ref[...], preferred_element_type=f32) + bih_ref[...]
    gh = jnp.dot(xb, whh_ref[...], preferred_element_type=f32) + bhh_ref[...]
    r = jax.nn.sigmoid(gi[:, :D] + gh[:, :D])
    z = jax.nn.sigmoid(gi[:, D:2 * D] + gh[:, D:2 * D])
    n = jnp.tanh(gi[:, 2 * D:] + r * gh[:, 2 * D:])
    hn = (1.0 - z) * n + z * xb
    hn_ref[...] = hn
    xo_ref[...] = jnp.maximum(hn + xb, 0.0)


def kernel(x, edge_index, edge_attr, W1, b1, W2, b2, W_root, b_conv,
           W_ih, W_hh, b_ih, b_hh):
    N, D = x.shape
    E = edge_index.shape[1]
    ED = edge_attr.shape[1]
    HNN = W1.shape[1]

    NCH = -(-E // (NW * C))
    EP = NW * NCH * C
    BN = 1000
    NP = BN
    while NP < N + 1 or NP % (NS * 8) != 0:
        NP += BN
    RPS = NP // NS
    pad = EP - E

    src = edge_index[0]
    dst = edge_index[1]
    src_p = jnp.concatenate([src, jnp.zeros((pad,), i32)]).reshape(NW, NCH, C)
    dst_p = jnp.concatenate([dst, jnp.full((pad,), N, i32)]).reshape(NW, NCH, C)

    x_j = _gather_kernel(N, D, NCH)(x.astype(bf16), src_p)

    W1T = W1.T
    b1c = b1[:, None]
    W2c = W2.reshape(HNN, D, D).transpose(2, 0, 1).reshape(D, HNN * D)
    W2full = jnp.concatenate([W2c, b2.reshape(D, D).T], axis=1)

    BE = 1280
    n_blk = E // BE
    msg_p = pl.pallas_call(
        functools.partial(_msg_body, ED, D, HNN),
        grid=(n_blk,),
        in_specs=[
            pl.BlockSpec((BE, ED), lambda i: (i, 0)),
            pl.BlockSpec((BE, D), lambda i: (i, 0)),
            pl.BlockSpec((HNN, ED), lambda i: (0, 0)),
            pl.BlockSpec((HNN, 1), lambda i: (0, 0)),
            pl.BlockSpec((D, HNN * D + D), lambda i: (0, 0)),
        ],
        out_specs=pl.BlockSpec((BE, D), lambda i: (i, 0)),
        out_shape=jax.ShapeDtypeStruct((EP, D), f32),
        scratch_shapes=[pltpu.VMEM((HNN * D + D, BE), bf16)],
    )(edge_attr, x_j, W1T, b1c, W2full.astype(bf16))

    z32 = jnp.zeros((RPS, D), f32)
    z16 = jnp.zeros((RPS, 16), f32)
    agg_out, cnt_out = _scatter_kernel(NP, D, NCH, RPS)(
        msg_p, dst_p, z32, z16)

    n_blk2 = N // BN
    npb = NP // BN
    x_out, h_new = pl.pallas_call(
        functools.partial(_tail_body, D),
        grid=(n_blk2,),
        in_specs=[
            pl.BlockSpec((BN, D), lambda i: (i, 0)),
            pl.BlockSpec((BN, D), lambda i: (i, 0)),
            pl.BlockSpec((BN, D), lambda i: (i + npb, 0)),
            pl.BlockSpec((BN, 16), lambda i: (i, 0)),
            pl.BlockSpec((BN, 16), lambda i: (i + npb, 0)),
            pl.BlockSpec((D, D), lambda i: (0, 0)),
            pl.BlockSpec((1, D), lambda i: (0, 0)),
            pl.BlockSpec((D, 3 * D), lambda i: (0, 0)),
            pl.BlockSpec((1, 3 * D), lambda i: (0, 0)),
            pl.BlockSpec((D, 3 * D), lambda i: (0, 0)),
            pl.BlockSpec((1, 3 * D), lambda i: (0, 0)),
        ],
        out_specs=[
            pl.BlockSpec((BN, D), lambda i: (i, 0)),
            pl.BlockSpec((BN, D), lambda i: (i, 0)),
        ],
        out_shape=[
            jax.ShapeDtypeStruct((N, D), f32),
            jax.ShapeDtypeStruct((N, D), f32),
        ],
    )(x, agg_out, agg_out, cnt_out, cnt_out, W_root, b_conv[None, :],
      W_ih.T, b_ih[None, :], W_hh.T, b_hh[None, :])

    return (x_out, h_new[None, :, :])
